# baseline (speedup 1.0000x reference)
import functools

import jax
import jax.numpy as jnp
from jax import lax
from jax.experimental import pallas as pl
from jax.experimental.pallas import tpu as pltpu
from jax.experimental.pallas import tpu_sc as plsc

NUM_TASKS = 3
PROMPT_LEN = 20
HIDDEN = 4096
BATCH = 1024

WIDE = 2048
R_EL = PROMPT_LEN * HIDDEN // WIDE
ROWS = BATCH * R_EL
T_ROWS = NUM_TASKS * R_EL

NUM_CORES = 2
NUM_SUBCORES = 16
NUM_WORKERS = NUM_CORES * NUM_SUBCORES

B_PER_TILE = BATCH // NUM_WORKERS
N_B = 14
ROWS_B = N_B * R_EL
CHUNK = 8
N_CHUNKS = ROWS_B // CHUNK


def _sc_lookup(task_ids, row_idx, table2):
    mesh = plsc.VectorSubcoreMesh(core_axis_name="c", subcore_axis_name="s")

    @functools.partial(
        pl.kernel,
        out_type=jax.ShapeDtypeStruct((ROWS, WIDE), jnp.float32),
        mesh=mesh,
        scratch_types=[
            pltpu.VMEM((B_PER_TILE,), jnp.int32),
            pltpu.VMEM((ROWS_B,), jnp.int32),
            pltpu.VMEM((CHUNK, WIDE), jnp.float32),
            pltpu.VMEM((CHUNK, WIDE), jnp.float32),
            pltpu.VMEM_SHARED((T_ROWS, WIDE), jnp.float32),
            pltpu.SemaphoreType.DMA,
            pltpu.SemaphoreType.DMA,
            pltpu.SemaphoreType.DMA,
            pltpu.SemaphoreType.DMA,
            pltpu.SemaphoreType.DMA,
        ],
    )
    def run(ids_hbm, ridx_hbm, table2_hbm, out_hbm,
            idx_a, idx_b, buf0, buf1, sh_table,
            asem, gsem0, gsem1, ssem0, ssem1):
        c = lax.axis_index("c")
        s = lax.axis_index("s")
        wid = s * NUM_CORES + c
        base_el = wid * B_PER_TILE
        base_row = base_el * R_EL

        pltpu.sync_copy(ids_hbm.at[pl.ds(base_el, B_PER_TILE)], idx_a)
        pltpu.sync_copy(ridx_hbm.at[pl.ds(base_row, ROWS_B)], idx_b)

        @pl.when(s == 0)
        def _():
            pltpu.sync_copy(table2_hbm, sh_table)

        plsc.subcore_barrier()

        n_a = 0
        for g in range(B_PER_TILE // 16):
            vec = idx_a[pl.ds(g * 16, 16)]
            for i in range(16):
                e = g * 16 + i
                if e < N_B:
                    continue
                tid = vec[i]
                pltpu.async_copy(
                    sh_table.at[pl.ds(tid * R_EL, R_EL)],
                    out_hbm.at[pl.ds(base_row + e * R_EL, R_EL)],
                    asem)
                n_a += 1

        def gather(ch, buf, sem):
            pltpu.async_copy(
                table2_hbm.at[idx_b.at[pl.ds(ch * CHUNK, CHUNK)]], buf, sem)

        def gather_wait(buf, sem):
            pltpu.make_async_copy(
                table2_hbm.at[idx_b.at[pl.ds(0, CHUNK)]], buf, sem).wait()

        def scatter(ch, buf, sem):
            pltpu.async_copy(
                buf, out_hbm.at[pl.ds(base_row + ch * CHUNK, CHUNK)], sem)

        def scatter_wait(buf, sem):
            pltpu.make_async_copy(
                buf, out_hbm.at[pl.ds(base_row, CHUNK)], sem).wait()

        gather(0, buf0, gsem0)
        gather(1, buf1, gsem1)
        last = N_CHUNKS - 1

        @pl.loop(0, N_CHUNKS, step=2)
        def _(g):
            gather_wait(buf0, gsem0)
            scatter(g, buf0, ssem0)
            gather_wait(buf1, gsem1)
            scatter(g + 1, buf1, ssem1)
            scatter_wait(buf0, ssem0)
            gather(jnp.minimum(g + 2, last), buf0, gsem0)
            scatter_wait(buf1, ssem1)
            gather(jnp.minimum(g + 3, last), buf1, gsem1)

        gather_wait(buf0, gsem0)
        gather_wait(buf1, gsem1)

        for _ in range(n_a):
            pltpu.make_async_copy(
                sh_table.at[pl.ds(0, R_EL)],
                out_hbm.at[pl.ds(base_row, R_EL)], asem).wait()

    return run(task_ids, row_idx, table2)


def kernel(task_ids, prompt_embeddings):
    ids = task_ids.astype(jnp.int32)
    row_idx = (ids[:, None] * R_EL
               + jnp.arange(R_EL, dtype=jnp.int32)).reshape(ROWS)
    table2 = prompt_embeddings.reshape(T_ROWS, WIDE)
    out = _sc_lookup(ids, row_idx, table2)
    return out.reshape(BATCH, PROMPT_LEN, HIDDEN)


# speedup vs baseline: 1.1451x; 1.1451x over previous
import functools

import jax
import jax.numpy as jnp
from jax import lax
from jax.experimental import pallas as pl
from jax.experimental.pallas import tpu as pltpu
from jax.experimental.pallas import tpu_sc as plsc

NUM_TASKS = 3
PROMPT_LEN = 20
HIDDEN = 4096
BATCH = 1024

NUM_CORES = 2
NUM_SUBCORES = 16
NUM_WORKERS = NUM_CORES * NUM_SUBCORES

B_PER_TILE = 2 * BATCH // NUM_WORKERS
FLIGHT = 16


def _sc_lookup(task_ids, table):
    mesh = plsc.VectorSubcoreMesh(core_axis_name="c", subcore_axis_name="s")

    @functools.partial(
        pl.kernel,
        out_type=jax.ShapeDtypeStruct((BATCH, PROMPT_LEN, HIDDEN), jnp.float32),
        mesh=mesh,
        scratch_types=[
            pltpu.VMEM((B_PER_TILE,), jnp.int32),
            pltpu.VMEM_SHARED((NUM_TASKS, PROMPT_LEN, HIDDEN), jnp.float32),
            pltpu.SemaphoreType.DMA,
        ],
    )
    def run(idx_hbm, table_hbm, out_hbm, idx_v, sh_table, sem):
        c = lax.axis_index("c")
        s = lax.axis_index("s")
        wid = s * NUM_CORES + c

        @pl.when(s == 0)
        def _():
            pltpu.sync_copy(table_hbm, sh_table)

        plsc.subcore_barrier()

        @pl.when(wid % 2 == 0)
        def _():
            base = wid * (B_PER_TILE // 2)
            pltpu.sync_copy(idx_hbm.at[pl.ds(base, B_PER_TILE)], idx_v)

            def wait_one():
                pltpu.make_async_copy(
                    sh_table.at[0], out_hbm.at[base], sem).wait()

            inflight = 0
            for g in range(B_PER_TILE // 16):
                vec = idx_v[pl.ds(g * 16, 16)]
                for i in range(16):
                    tid = vec[i]
                    pltpu.async_copy(
                        sh_table.at[tid], out_hbm.at[base + g * 16 + i], sem)
                    inflight += 1
                    if inflight >= FLIGHT:
                        wait_one()
                        inflight -= 1
            for _ in range(inflight):
                wait_one()

    return run(task_ids, table)


def kernel(task_ids, prompt_embeddings):
    return _sc_lookup(task_ids.astype(jnp.int32), prompt_embeddings)
